# baseline (speedup 1.0000x reference)
import dataclasses

import jax
import jax.numpy as jnp
from jax import lax
from jax.experimental import pallas as pl
from jax.experimental.pallas import tpu as pltpu
from jax.experimental.pallas import tpu_sc as plsc

N = 10000
E = 320000
D = 128
H = 3
U = 128
ROW = 144
NC = 2
NS = 16
NW = NC * NS
EPT = E // NW
BLK = 32
KR = 4
NB = KR * ((EPT + KR * BLK - 1) // (KR * BLK))
EPT_PAD = NB * BLK
SH_ROWS = 10240
STRIPE = SH_ROWS // NS
LANES = 16


def _proj_kernel(x_ref, w_ref, a1_ref, a2_ref, hh_ref, p2_ref):
    xb = x_ref[...]
    w = w_ref[...]
    hb = jnp.dot(xb, w, preferred_element_type=jnp.float32)
    w3 = w.reshape(D, H, U)
    q1 = jnp.sum(w3 * a1_ref[...].reshape(1, H, U), axis=-1)
    q2 = jnp.sum(w3 * a2_ref[...].reshape(1, H, U), axis=-1)
    p2_ref[...] = jnp.dot(xb, q2, preferred_element_type=jnp.float32)
    bn = xb.shape[0]
    p1b = jnp.dot(xb, q1, preferred_element_type=jnp.float32)
    ones = jnp.ones((bn, 1), jnp.float32)
    zpad = jnp.zeros((bn, ROW - U - 2), jnp.float32)
    for hd in range(H):
        hh_ref[hd] = jnp.concatenate(
            [hb[:, hd * U:(hd + 1) * U], ones, p1b[:, hd:hd + 1], zpad],
            axis=1)


def _stage1(x, W, a1, a2):
    bn = 1000
    return pl.pallas_call(
        _proj_kernel,
        grid=(N // bn,),
        in_specs=[
            pl.BlockSpec((bn, D), lambda i: (i, 0)),
            pl.BlockSpec((D, H * U), lambda i: (0, 0)),
            pl.BlockSpec((H, 1, U), lambda i: (0, 0, 0)),
            pl.BlockSpec((H, 1, U), lambda i: (0, 0, 0)),
        ],
        out_specs=[
            pl.BlockSpec((H, bn, ROW), lambda i: (0, i, 0)),
            pl.BlockSpec((bn, H), lambda i: (i, 0)),
        ],
        out_shape=[
            jax.ShapeDtypeStruct((H, N, ROW), jnp.float32),
            jax.ShapeDtypeStruct((N, H), jnp.float32),
        ],
    )(x, W, a1, a2)


def _sc_kernel(hh_hbm, src_hbm, dst_hbm, p2_hbm, part_hbm,
               srcb0_v, srcb1_v, dstb0_v, dstb1_v,
               sidx0_v, sidx1_v, sidx2_v, sidx3_v,
               rows0_v, rows1_v, rows2_v, rows3_v, att_v,
               idx0_v, idx1_v, idx2_v, idx3_v,
               p2b0_v, p2b1_v, p2b2_v, p2b3_v,
               p2i0_v, p2i1_v, p2i2_v, p2i3_v,
               out_sh, seme0, seme1,
               semg0, semg1, semg2, semg3,
               semp0, semp1, semp2, semp3, semw):
    cid = lax.axis_index("c")
    sid = lax.axis_index("s")
    wid = cid * NS + sid
    ebase = wid * EPT_PAD

    src_bufs = (srcb0_v, srcb1_v)
    dst_bufs = (dstb0_v, dstb1_v)
    sidx_bufs = (sidx0_v, sidx1_v, sidx2_v, sidx3_v)
    rows_bufs = (rows0_v, rows1_v, rows2_v, rows3_v)
    idx_bufs = (idx0_v, idx1_v, idx2_v, idx3_v)
    p2_bufs = (p2b0_v, p2b1_v, p2b2_v, p2b3_v)
    p2i_bufs = (p2i0_v, p2i1_v, p2i2_v, p2i3_v)
    semes = (seme0, seme1)
    sems = (semg0, semg1, semg2, semg3)
    semps = (semp0, semp1, semp2, semp3)

    if True:
        def zero_rows0():
            @pl.loop(0, BLK)
            def _(e):
                for c in range(ROW // LANES):
                    rows0_v[e, pl.ds(c * LANES, LANES)] = jnp.zeros(
                        (LANES,), jnp.float32)

        def zero_out_sh():
            base = sid * STRIPE
            for k in range(STRIPE // BLK):
                pltpu.sync_copy(rows0_v,
                                out_sh.at[pl.ds(base + k * BLK, BLK)])

        def load_edges(b, eb):
            off = pl.ds(ebase + b * BLK, BLK)
            pltpu.async_copy(src_hbm.at[off], src_bufs[eb], semes[eb])
            pltpu.async_copy(dst_hbm.at[off], dst_bufs[eb], semes[eb])

        def wait_edges(b, eb):
            off = pl.ds(ebase + b * BLK, BLK)
            pltpu.make_async_copy(src_hbm.at[off], src_bufs[eb],
                                  semes[eb]).wait()
            pltpu.make_async_copy(dst_hbm.at[off], dst_bufs[eb],
                                  semes[eb]).wait()

        def gather_block(hd, eb, rb):
            for s in range(BLK // LANES):
                sl = pl.ds(s * LANES, LANES)
                s16 = src_bufs[eb][sl]
                d16 = dst_bufs[eb][sl]
                idx_bufs[rb][sl] = s16 + hd * N
                p2i_bufs[rb][sl] = d16 * H + hd
                sidx_bufs[rb][sl] = d16
            pltpu.async_copy(hh_hbm.at[idx_bufs[rb]], rows_bufs[rb],
                             sems[rb])
            pltpu.async_copy(p2_hbm.at[p2i_bufs[rb]], p2_bufs[rb],
                             semps[rb])

        def wait_gather(rb):
            pltpu.make_async_copy(hh_hbm.at[idx_bufs[rb]], rows_bufs[rb],
                                  sems[rb]).wait()
            pltpu.make_async_copy(p2_hbm.at[p2i_bufs[rb]], p2_bufs[rb],
                                  semps[rb]).wait()

        def process_block(b, rb):
            rows = rows_bufs[rb]
            p2b = p2_bufs[rb]
            for s in range(BLK // LANES):
                e16 = s * LANES + lax.iota(jnp.int32, LANES)
                p1v = plsc.load_gather(
                    rows, [e16, jnp.full((LANES,), U + 1, jnp.int32)])
                lv = p1v + p2b[pl.ds(s * LANES, LANES)]
                lv = jnp.where(lv >= 0.0, lv, 0.2 * lv)
                av = jnp.exp(lv)
                j = b * BLK + s * LANES + lax.iota(jnp.int32, LANES)
                av = jnp.where(j < EPT, av, 0.0)
                att_v[pl.ds(s * LANES, LANES)] = av

            @plsc.parallel_loop(0, BLK, unroll=4)
            def _(e):
                a16 = plsc.load_gather(
                    att_v, [jnp.full((LANES,), e, jnp.int32)])
                for c in range(ROW // LANES):
                    sl = pl.ds(c * LANES, LANES)
                    rows[e, sl] = rows[e, sl] * a16

            pltpu.sync_copy(rows, out_sh.at[sidx_bufs[rb]], add=True)

        for hd in range(H):
            zero_rows0()
            zero_out_sh()
            plsc.subcore_barrier()

            load_edges(0, 0)
            for k in range(KR - 1):
                wait_edges(k, k % 2)
                gather_block(hd, k % 2, k)
                load_edges(k + 1, (k + 1) % 2)

            @pl.loop(0, NB // KR)
            def _(i):
                b0 = i * KR
                for k in range(KR):
                    b = b0 + k
                    g = b + KR - 1

                    @pl.when(g < NB)
                    def _():
                        wait_edges(g, (k + 1) % 2)
                        gather_block(hd, (k + 1) % 2, (k + KR - 1) % KR)

                    @pl.when(g + 1 < NB)
                    def _():
                        load_edges(g + 1, k % 2)

                    wait_gather(k)
                    process_block(b, k)

            plsc.subcore_barrier()
            base = sid * STRIPE
            pltpu.async_copy(
                out_sh.at[pl.ds(base, STRIPE)],
                part_hbm.at[cid, hd, pl.ds(base, STRIPE)],
                semw).wait()
            plsc.subcore_barrier()


def _stage2(hh_flat, src2d, dst2d, p2f):
    mesh = plsc.VectorSubcoreMesh(core_axis_name="c", subcore_axis_name="s")
    cp = pltpu.CompilerParams()
    if "needs_layout_passes" in pltpu.CompilerParams.__dataclass_fields__:
        cp = dataclasses.replace(cp, needs_layout_passes=False)
    if "use_tc_tiling_on_sc" in pltpu.CompilerParams.__dataclass_fields__:
        cp = dataclasses.replace(cp, use_tc_tiling_on_sc=False)
    kern = pl.kernel(
        _sc_kernel,
        out_type=jax.ShapeDtypeStruct((NC, H, SH_ROWS, ROW), jnp.float32),
        mesh=mesh,
        compiler_params=cp,
        scratch_types=(
            [pltpu.VMEM((BLK,), jnp.int32)] * 4
            + [pltpu.VMEM((BLK,), jnp.int32)] * KR
            + [pltpu.VMEM((BLK, ROW), jnp.float32)] * KR
            + [pltpu.VMEM((BLK,), jnp.float32)]
            + [pltpu.VMEM((BLK,), jnp.int32)] * KR
            + [pltpu.VMEM((BLK,), jnp.float32)] * KR
            + [pltpu.VMEM((BLK,), jnp.int32)] * KR
            + [pltpu.VMEM_SHARED((SH_ROWS, ROW), jnp.float32)]
            + [pltpu.SemaphoreType.DMA] * (2 + KR + KR + 1)
        ),
    )
    return kern(hh_flat, src2d, dst2d, p2f)


def _combine_kernel(part_ref, out_ref):
    o = part_ref[0] + part_ref[1]
    den = o[:, :, U:U + 1]
    val = o[:, :, 0:U]
    safe = den > 0.0
    r = val / jnp.where(safe, den, 1.0)
    r = jnp.where(safe, r, 0.0)
    r = jnp.where(r > 0.0, r, jnp.exp(r) - 1.0)
    for hd in range(H):
        out_ref[:, hd * U:(hd + 1) * U] = r[hd]


def _stage3(partials):
    bn = 1000
    return pl.pallas_call(
        _combine_kernel,
        grid=(N // bn,),
        in_specs=[pl.BlockSpec((NC, H, bn, ROW), lambda i: (0, 0, i, 0))],
        out_specs=pl.BlockSpec((bn, H * U), lambda i: (i, 0)),
        out_shape=jax.ShapeDtypeStruct((N, H * U), jnp.float32),
    )(partials)


@jax.jit
def kernel(x, edge_index, W, a1, a2):
    hh, p2 = _stage1(x, W, a1, a2)
    hh_flat = hh.reshape(H * N, ROW)
    p2f = p2.reshape(-1)

    src2d = jnp.pad(edge_index[:, 0].reshape(NW, EPT),
                    ((0, 0), (0, EPT_PAD - EPT))).reshape(-1)
    dst2d = jnp.pad(edge_index[:, 1].reshape(NW, EPT),
                    ((0, 0), (0, EPT_PAD - EPT))).reshape(-1)

    partials = _stage2(hh_flat, src2d, dst2d, p2f)
    return _stage3(partials)


# speedup vs baseline: 1.1002x; 1.1002x over previous
import dataclasses

import jax
import jax.numpy as jnp
from jax import lax
from jax.experimental import pallas as pl
from jax.experimental.pallas import tpu as pltpu
from jax.experimental.pallas import tpu_sc as plsc

N = 10000
E = 320000
D = 128
H = 3
U = 128
ROW = 144
NC = 2
NS = 16
NW = NC * NS
EPT = E // NW
BLK = 32
NB = 2 * ((EPT + 2 * BLK - 1) // (2 * BLK))
EPT_PAD = NB * BLK
SH_ROWS = 10240
STRIPE = SH_ROWS // NS
LANES = 16


def _proj_kernel(x_ref, w_ref, a1_ref, a2_ref, hh_ref, p2_ref):
    xb = x_ref[...]
    w = w_ref[...]
    hb = jnp.dot(xb, w, preferred_element_type=jnp.float32)
    w3 = w.reshape(D, H, U)
    q1 = jnp.sum(w3 * a1_ref[...].reshape(1, H, U), axis=-1)
    q2 = jnp.sum(w3 * a2_ref[...].reshape(1, H, U), axis=-1)
    p2_ref[...] = jnp.dot(xb, q2, preferred_element_type=jnp.float32)
    bn = xb.shape[0]
    p1b = jnp.dot(xb, q1, preferred_element_type=jnp.float32)
    ones = jnp.ones((bn, 1), jnp.float32)
    zpad = jnp.zeros((bn, ROW - U - 2), jnp.float32)
    for hd in range(H):
        hh_ref[hd] = jnp.concatenate(
            [hb[:, hd * U:(hd + 1) * U], ones, p1b[:, hd:hd + 1], zpad],
            axis=1)


def _stage1(x, W, a1, a2):
    bn = 1000
    return pl.pallas_call(
        _proj_kernel,
        grid=(N // bn,),
        in_specs=[
            pl.BlockSpec((bn, D), lambda i: (i, 0)),
            pl.BlockSpec((D, H * U), lambda i: (0, 0)),
            pl.BlockSpec((H, 1, U), lambda i: (0, 0, 0)),
            pl.BlockSpec((H, 1, U), lambda i: (0, 0, 0)),
        ],
        out_specs=[
            pl.BlockSpec((H, bn, ROW), lambda i: (0, i, 0)),
            pl.BlockSpec((bn, H), lambda i: (i, 0)),
        ],
        out_shape=[
            jax.ShapeDtypeStruct((H, N, ROW), jnp.float32),
            jax.ShapeDtypeStruct((N, H), jnp.float32),
        ],
    )(x, W, a1, a2)


def _sc_kernel(hh_hbm, src_hbm, dst_hbm, p2_hbm, part_hbm,
               srcb0_v, srcb1_v, dstb0_v, dstb1_v, sidx0_v, sidx1_v,
               rows0_v, rows1_v, att_v,
               idx0_v, idx1_v, p2b0_v, p2b1_v, p2i0_v, p2i1_v,
               out_sh, seme0, seme1, sem0, sem1, semp0, semp1,
               semsc0, semsc1, semw):
    cid = lax.axis_index("c")
    sid = lax.axis_index("s")
    wid = cid * NS + sid
    ebase = wid * EPT_PAD

    src_bufs = (srcb0_v, srcb1_v)
    dst_bufs = (dstb0_v, dstb1_v)
    sidx_bufs = (sidx0_v, sidx1_v)
    rows_bufs = (rows0_v, rows1_v)
    idx_bufs = (idx0_v, idx1_v)
    p2_bufs = (p2b0_v, p2b1_v)
    p2i_bufs = (p2i0_v, p2i1_v)
    semes = (seme0, seme1)
    sems = (sem0, sem1)
    semps = (semp0, semp1)
    semscs = (semsc0, semsc1)

    if True:
        def zero_rows0():
            @pl.loop(0, BLK)
            def _(e):
                for c in range(ROW // LANES):
                    rows0_v[e, pl.ds(c * LANES, LANES)] = jnp.zeros(
                        (LANES,), jnp.float32)

        def zero_out_sh():
            base = sid * STRIPE
            for k in range(STRIPE // BLK):
                pltpu.sync_copy(rows0_v,
                                out_sh.at[pl.ds(base + k * BLK, BLK)])

        def load_edges(b, eb):
            off = pl.ds(ebase + b * BLK, BLK)
            pltpu.async_copy(src_hbm.at[off], src_bufs[eb], semes[eb])
            pltpu.async_copy(dst_hbm.at[off], dst_bufs[eb], semes[eb])

        def wait_edges(b, eb):
            off = pl.ds(ebase + b * BLK, BLK)
            pltpu.make_async_copy(src_hbm.at[off], src_bufs[eb],
                                  semes[eb]).wait()
            pltpu.make_async_copy(dst_hbm.at[off], dst_bufs[eb],
                                  semes[eb]).wait()

        def wait_scatter(rb):
            pltpu.make_async_copy(rows_bufs[rb], out_sh.at[sidx_bufs[rb]],
                                  semscs[rb]).wait()

        def gather_block(hd, eb, rb):
            for s in range(BLK // LANES):
                sl = pl.ds(s * LANES, LANES)
                s16 = src_bufs[eb][sl]
                d16 = dst_bufs[eb][sl]
                idx_bufs[rb][sl] = s16 + hd * N
                p2i_bufs[rb][sl] = d16 * H + hd
                sidx_bufs[rb][sl] = d16
            pltpu.async_copy(hh_hbm.at[idx_bufs[rb]], rows_bufs[rb],
                             sems[rb])
            pltpu.async_copy(p2_hbm.at[p2i_bufs[rb]], p2_bufs[rb],
                             semps[rb])

        def wait_gather(rb):
            pltpu.make_async_copy(hh_hbm.at[idx_bufs[rb]], rows_bufs[rb],
                                  sems[rb]).wait()
            pltpu.make_async_copy(p2_hbm.at[p2i_bufs[rb]], p2_bufs[rb],
                                  semps[rb]).wait()

        def process_block(b, rb):
            rows = rows_bufs[rb]
            p2b = p2_bufs[rb]
            for s in range(BLK // LANES):
                e16 = s * LANES + lax.iota(jnp.int32, LANES)
                p1v = plsc.load_gather(
                    rows, [e16, jnp.full((LANES,), U + 1, jnp.int32)])
                lv = p1v + p2b[pl.ds(s * LANES, LANES)]
                lv = jnp.where(lv >= 0.0, lv, 0.2 * lv)
                av = jnp.exp(lv)
                j = b * BLK + s * LANES + lax.iota(jnp.int32, LANES)
                av = jnp.where(j < EPT, av, 0.0)
                att_v[pl.ds(s * LANES, LANES)] = av

            @plsc.parallel_loop(0, BLK, unroll=4)
            def _(e):
                a16 = plsc.load_gather(
                    att_v, [jnp.full((LANES,), e, jnp.int32)])
                for c in range(ROW // LANES):
                    sl = pl.ds(c * LANES, LANES)
                    rows[e, sl] = rows[e, sl] * a16

            pltpu.async_copy(rows, out_sh.at[sidx_bufs[rb]], semscs[rb],
                             add=True)

        for hd in range(H):
            zero_rows0()
            zero_out_sh()
            plsc.subcore_barrier()

            load_edges(0, 0)
            wait_edges(0, 0)
            gather_block(hd, 0, 0)
            load_edges(1, 1)

            @pl.loop(0, NB // 2)
            def _(i):
                b0 = i * 2
                wait_edges(b0 + 1, 1)

                @pl.when(b0 >= 1)
                def _():
                    wait_scatter(1)

                gather_block(hd, 1, 1)

                @pl.when(b0 + 2 < NB)
                def _():
                    load_edges(b0 + 2, 0)

                wait_gather(0)
                process_block(b0, 0)

                @pl.when(b0 + 2 < NB)
                def _():
                    wait_edges(b0 + 2, 0)
                    wait_scatter(0)
                    gather_block(hd, 0, 0)

                @pl.when(b0 + 3 < NB)
                def _():
                    load_edges(b0 + 3, 1)

                wait_gather(1)
                process_block(b0 + 1, 1)

            wait_scatter(0)
            wait_scatter(1)
            plsc.subcore_barrier()
            base = sid * STRIPE
            pltpu.async_copy(
                out_sh.at[pl.ds(base, STRIPE)],
                part_hbm.at[cid, hd, pl.ds(base, STRIPE)],
                semw).wait()
            plsc.subcore_barrier()


def _stage2(hh_flat, src2d, dst2d, p2f):
    mesh = plsc.VectorSubcoreMesh(core_axis_name="c", subcore_axis_name="s")
    cp = pltpu.CompilerParams()
    if "needs_layout_passes" in pltpu.CompilerParams.__dataclass_fields__:
        cp = dataclasses.replace(cp, needs_layout_passes=False)
    if "use_tc_tiling_on_sc" in pltpu.CompilerParams.__dataclass_fields__:
        cp = dataclasses.replace(cp, use_tc_tiling_on_sc=False)
    kern = pl.kernel(
        _sc_kernel,
        out_type=jax.ShapeDtypeStruct((NC, H, SH_ROWS, ROW), jnp.float32),
        mesh=mesh,
        compiler_params=cp,
        scratch_types=[
            pltpu.VMEM((BLK,), jnp.int32),
            pltpu.VMEM((BLK,), jnp.int32),
            pltpu.VMEM((BLK,), jnp.int32),
            pltpu.VMEM((BLK,), jnp.int32),
            pltpu.VMEM((BLK,), jnp.int32),
            pltpu.VMEM((BLK,), jnp.int32),
            pltpu.VMEM((BLK, ROW), jnp.float32),
            pltpu.VMEM((BLK, ROW), jnp.float32),
            pltpu.VMEM((BLK,), jnp.float32),
            pltpu.VMEM((BLK,), jnp.int32),
            pltpu.VMEM((BLK,), jnp.int32),
            pltpu.VMEM((BLK,), jnp.float32),
            pltpu.VMEM((BLK,), jnp.float32),
            pltpu.VMEM((BLK,), jnp.int32),
            pltpu.VMEM((BLK,), jnp.int32),
            pltpu.VMEM_SHARED((SH_ROWS, ROW), jnp.float32),
            pltpu.SemaphoreType.DMA,
            pltpu.SemaphoreType.DMA,
            pltpu.SemaphoreType.DMA,
            pltpu.SemaphoreType.DMA,
            pltpu.SemaphoreType.DMA,
            pltpu.SemaphoreType.DMA,
            pltpu.SemaphoreType.DMA,
            pltpu.SemaphoreType.DMA,
            pltpu.SemaphoreType.DMA,
        ],
    )
    return kern(hh_flat, src2d, dst2d, p2f)


def _combine_kernel(part_ref, out_ref):
    o = part_ref[0] + part_ref[1]
    den = o[:, :, U:U + 1]
    val = o[:, :, 0:U]
    safe = den > 0.0
    r = val / jnp.where(safe, den, 1.0)
    r = jnp.where(safe, r, 0.0)
    r = jnp.where(r > 0.0, r, jnp.exp(r) - 1.0)
    for hd in range(H):
        out_ref[:, hd * U:(hd + 1) * U] = r[hd]


def _stage3(partials):
    bn = 1000
    return pl.pallas_call(
        _combine_kernel,
        grid=(N // bn,),
        in_specs=[pl.BlockSpec((NC, H, bn, ROW), lambda i: (0, 0, i, 0))],
        out_specs=pl.BlockSpec((bn, H * U), lambda i: (i, 0)),
        out_shape=jax.ShapeDtypeStruct((N, H * U), jnp.float32),
    )(partials)


@jax.jit
def kernel(x, edge_index, W, a1, a2):
    hh, p2 = _stage1(x, W, a1, a2)
    hh_flat = hh.reshape(H * N, ROW)
    p2f = p2.reshape(-1)

    src2d = jnp.pad(edge_index[:, 0].reshape(NW, EPT),
                    ((0, 0), (0, EPT_PAD - EPT))).reshape(-1)
    dst2d = jnp.pad(edge_index[:, 1].reshape(NW, EPT),
                    ((0, 0), (0, EPT_PAD - EPT))).reshape(-1)

    partials = _stage2(hh_flat, src2d, dst2d, p2f)
    return _stage3(partials)
